# baseline (speedup 1.0000x reference)
import jax
import jax.numpy as jnp
from jax.experimental import pallas as pl
from jax.experimental.pallas import tpu as pltpu

NGRAM = 3
C = 4
LEVELS = 256
D = 8192
T = 1024
TB = 256
NTB = T // TB

_BIG_USED = (0, 2, 3, 4)
_SMALL_USED = (0, 4, 5, 6, 3, 17, 11, 12, 15)


def _prebind_kernel(lw_ref, keys_ref, m_ref):
    lw = lw_ref[...]
    for c in range(C):
        m_ref[c * LEVELS:(c + 1) * LEVELS, :] = lw * keys_ref[c][None, :]


def _stageA_kernel(sig_ref, m_ref, out_ref, prev2_ref):
    i = pl.program_id(0)
    idx = jnp.clip(jnp.round(sig_ref[...] * (LEVELS - 1)).astype(jnp.int32),
                   0, LEVELS - 1)
    iota_l = jax.lax.broadcasted_iota(jnp.int32, (TB, LEVELS), 1)
    onehot = jnp.concatenate(
        [(idx[:, c][:, None] == iota_l) for c in range(C)],
        axis=1).astype(jnp.bfloat16)
    acc = jax.lax.dot_general(onehot, m_ref[...],
                              (((1,), (0,)), ((), ())),
                              preferred_element_type=jnp.float32)
    acc = acc.astype(jnp.bfloat16)
    rows = jnp.concatenate([prev2_ref[...], acc], axis=0)
    a = rows[0:TB]
    b = rows[1:TB + 1]
    cc = rows[2:TB + 2]
    a2 = jnp.concatenate([a[:, -2:], a[:, :-2]], axis=1)
    b1 = jnp.concatenate([b[:, -1:], b[:, :-1]], axis=1)
    prod = a2 * b1 * cc
    nskip = jnp.where(i == 0, 2, 0)
    ones_mask = (jax.lax.broadcasted_iota(jnp.int32, (1, TB), 1)
                 >= nskip).astype(jnp.bfloat16)
    part = jax.lax.dot_general(ones_mask, prod, (((1,), (0,)), ((), ())),
                               preferred_element_type=jnp.float32)

    @pl.when(i == 0)
    def _():
        out_ref[...] = jnp.zeros_like(out_ref)

    out_ref[...] += part
    prev2_ref[...] = acc[TB - 2:TB]


def _stageB_kernel(sample_ref, wb_ref, ws_ref, fb_ref, bb_ref, fs_ref, bs_ref,
                   out_ref, mprod_ref, sa_ref, sb_ref):
    j = pl.program_id(0)

    @pl.when(j == 0)
    def _():
        mprod_ref[...] = jnp.ones_like(mprod_ref)
        sa_ref[...] = jnp.zeros_like(sa_ref)
        sb_ref[...] = jnp.zeros_like(sb_ref)

    def hv(w, fcol, brow):
        wb = w.astype(jnp.float32)
        fc = fcol.astype(jnp.bfloat16).astype(jnp.float32)
        p = jnp.sum(wb * fc, axis=0)[None, :]
        return jnp.cos(p + brow) * jnp.sin(p)

    @pl.when(j < 4)
    def _():
        mprod_ref[...] *= hv(wb_ref[0], fb_ref[0], bb_ref[0])

    @pl.when(jnp.logical_and(j >= 4, j < 8))
    def _():
        sa_ref[...] += hv(ws_ref[0], fs_ref[0], bs_ref[0])

    @pl.when(jnp.logical_and(j >= 8, j < 12))
    def _():
        sb_ref[...] += hv(ws_ref[0], fs_ref[0], bs_ref[0])

    @pl.when(j == 12)
    def _():
        h21 = hv(ws_ref[0], fs_ref[0], bs_ref[0])
        mult = mprod_ref[...] * sa_ref[...] * sb_ref[...] * h21
        s = sample_ref[...] * mult
        out_ref[...] = jnp.where(s > 0, 1.0, -1.0).astype(jnp.float32)


def kernel(signals, feat, keys_hv, level_weight, W_big, b_big, W_small, b_small):
    lw = level_weight.astype(jnp.bfloat16)
    keys = keys_hv.astype(jnp.bfloat16)

    m_tab = pl.pallas_call(
        _prebind_kernel,
        in_specs=[
            pl.BlockSpec((LEVELS, D), lambda: (0, 0)),
            pl.BlockSpec((C, D), lambda: (0, 0)),
        ],
        out_specs=pl.BlockSpec((C * LEVELS, D), lambda: (0, 0)),
        out_shape=jax.ShapeDtypeStruct((C * LEVELS, D), jnp.bfloat16),
    )(lw, keys)

    sample = pl.pallas_call(
        _stageA_kernel,
        grid=(NTB,),
        in_specs=[
            pl.BlockSpec((TB, C), lambda i: (i, 0)),
            pl.BlockSpec((C * LEVELS, D), lambda i: (0, 0)),
        ],
        out_specs=pl.BlockSpec((1, D), lambda i: (0, 0)),
        out_shape=jax.ShapeDtypeStruct((1, D), jnp.float32),
        scratch_shapes=[pltpu.VMEM((2, D), jnp.bfloat16)],
    )(signals, m_tab)

    bigsel = jnp.array(_BIG_USED)
    smallsel = jnp.array(_SMALL_USED)
    wbT = W_big[bigsel].astype(jnp.bfloat16).transpose(0, 2, 1)
    wsT = W_small[smallsel].astype(jnp.bfloat16).transpose(0, 2, 1)
    fbT = feat[:546].reshape(6, 1, 91)[bigsel].transpose(0, 2, 1)
    fsT = feat[546:600].reshape(18, 1, 3)[smallsel].transpose(0, 2, 1)
    bb = b_big[bigsel][:, None, :]
    bs = b_small[smallsel][:, None, :]

    def wb_map(j):
        return (jnp.minimum(j, 3), 0, 0)

    def ws_map(j):
        return (jnp.maximum(j - 4, 0), 0, 0)

    out2d = pl.pallas_call(
        _stageB_kernel,
        grid=(13,),
        in_specs=[
            pl.BlockSpec((1, D), lambda j: (0, 0)),
            pl.BlockSpec((1, 91, D), wb_map),
            pl.BlockSpec((1, 3, D), ws_map),
            pl.BlockSpec((1, 91, 1), wb_map),
            pl.BlockSpec((1, 1, D), wb_map),
            pl.BlockSpec((1, 3, 1), ws_map),
            pl.BlockSpec((1, 1, D), ws_map),
        ],
        out_specs=pl.BlockSpec((1, D), lambda j: (0, 0)),
        out_shape=jax.ShapeDtypeStruct((1, D), jnp.float32),
        scratch_shapes=[
            pltpu.VMEM((1, D), jnp.float32),
            pltpu.VMEM((1, D), jnp.float32),
            pltpu.VMEM((1, D), jnp.float32),
        ],
    )(sample, wbT, wsT, fbT, bb, fsT, bs)

    return out2d.reshape(-1)


# speedup vs baseline: 1.0214x; 1.0214x over previous
import jax
import jax.numpy as jnp
from jax.experimental import pallas as pl
from jax.experimental.pallas import tpu as pltpu

NGRAM = 3
C = 4
LEVELS = 256
D = 8192
T = 1024
TB = 256
NTB = T // TB

_BIG_USED = (0, 2, 3, 4)
_SMALL_USED = (0, 4, 5, 6, 3, 17, 11, 12, 15)


def _prebind_kernel(lw_ref, keys_ref, m_ref):
    lw = lw_ref[...]
    for c in range(C):
        m_ref[c * LEVELS:(c + 1) * LEVELS, :] = lw * keys_ref[c][None, :]


def _stageA_kernel(sig_ref, m_ref, out_ref, prev2_ref):
    i = pl.program_id(0)
    idx = jnp.clip(jnp.round(sig_ref[...] * (LEVELS - 1)).astype(jnp.int32),
                   0, LEVELS - 1)
    iota_l = jax.lax.broadcasted_iota(jnp.int32, (TB, LEVELS), 1)
    onehot = jnp.concatenate(
        [(idx[:, c][:, None] == iota_l) for c in range(C)],
        axis=1).astype(jnp.bfloat16)
    acc = jax.lax.dot_general(onehot, m_ref[...],
                              (((1,), (0,)), ((), ())),
                              preferred_element_type=jnp.float32)
    acc = acc.astype(jnp.bfloat16)
    rows = jnp.concatenate([prev2_ref[...], acc], axis=0)
    a = rows[0:TB]
    b = rows[1:TB + 1]
    cc = rows[2:TB + 2]
    a2 = jnp.concatenate([a[:, -2:], a[:, :-2]], axis=1)
    b1 = jnp.concatenate([b[:, -1:], b[:, :-1]], axis=1)
    prod = (a2 * b1 * cc).astype(jnp.float32)
    nskip = jnp.where(i == 0, 2, 0)
    riota = jax.lax.broadcasted_iota(jnp.int32, (TB, D), 0)
    prod = jnp.where(riota >= nskip, prod, 0.0)
    part = jnp.sum(prod, axis=0, keepdims=True)

    @pl.when(i == 0)
    def _():
        out_ref[...] = jnp.zeros_like(out_ref)

    out_ref[...] += part
    prev2_ref[...] = acc[TB - 2:TB]


def _stageB_kernel(sample_ref, wb_ref, ws_ref, fb_ref, bb_ref, fs_ref, bs_ref,
                   out_ref, mprod_ref, sa_ref, sb_ref):
    j = pl.program_id(0)

    @pl.when(j == 0)
    def _():
        mprod_ref[...] = jnp.ones_like(mprod_ref)
        sa_ref[...] = jnp.zeros_like(sa_ref)
        sb_ref[...] = jnp.zeros_like(sb_ref)

    def hv(w, fcol, brow):
        wb = w.astype(jnp.float32)
        fc = fcol.astype(jnp.bfloat16).astype(jnp.float32)
        p = jnp.sum(wb * fc, axis=0)[None, :]
        return jnp.cos(p + brow) * jnp.sin(p)

    @pl.when(j < 4)
    def _():
        mprod_ref[...] *= hv(wb_ref[0], fb_ref[0], bb_ref[0])

    @pl.when(jnp.logical_and(j >= 4, j < 8))
    def _():
        sa_ref[...] += hv(ws_ref[0], fs_ref[0], bs_ref[0])

    @pl.when(jnp.logical_and(j >= 8, j < 12))
    def _():
        sb_ref[...] += hv(ws_ref[0], fs_ref[0], bs_ref[0])

    @pl.when(j == 12)
    def _():
        h21 = hv(ws_ref[0], fs_ref[0], bs_ref[0])
        mult = mprod_ref[...] * sa_ref[...] * sb_ref[...] * h21
        s = sample_ref[...] * mult
        out_ref[...] = jnp.where(s > 0, 1.0, -1.0).astype(jnp.float32)


def kernel(signals, feat, keys_hv, level_weight, W_big, b_big, W_small, b_small):
    lw = level_weight.astype(jnp.bfloat16)
    keys = keys_hv.astype(jnp.bfloat16)

    m_tab = pl.pallas_call(
        _prebind_kernel,
        in_specs=[
            pl.BlockSpec((LEVELS, D), lambda: (0, 0)),
            pl.BlockSpec((C, D), lambda: (0, 0)),
        ],
        out_specs=pl.BlockSpec((C * LEVELS, D), lambda: (0, 0)),
        out_shape=jax.ShapeDtypeStruct((C * LEVELS, D), jnp.bfloat16),
    )(lw, keys)

    sample = pl.pallas_call(
        _stageA_kernel,
        grid=(NTB,),
        in_specs=[
            pl.BlockSpec((TB, C), lambda i: (i, 0)),
            pl.BlockSpec((C * LEVELS, D), lambda i: (0, 0)),
        ],
        out_specs=pl.BlockSpec((1, D), lambda i: (0, 0)),
        out_shape=jax.ShapeDtypeStruct((1, D), jnp.float32),
        scratch_shapes=[pltpu.VMEM((2, D), jnp.bfloat16)],
    )(signals, m_tab)

    bigsel = jnp.array(_BIG_USED)
    smallsel = jnp.array(_SMALL_USED)
    wbT = W_big[bigsel].astype(jnp.bfloat16).transpose(0, 2, 1)
    wsT = W_small[smallsel].astype(jnp.bfloat16).transpose(0, 2, 1)
    fbT = feat[:546].reshape(6, 1, 91)[bigsel].transpose(0, 2, 1)
    fsT = feat[546:600].reshape(18, 1, 3)[smallsel].transpose(0, 2, 1)
    bb = b_big[bigsel][:, None, :]
    bs = b_small[smallsel][:, None, :]

    def wb_map(j):
        return (jnp.minimum(j, 3), 0, 0)

    def ws_map(j):
        return (jnp.maximum(j - 4, 0), 0, 0)

    out2d = pl.pallas_call(
        _stageB_kernel,
        grid=(13,),
        in_specs=[
            pl.BlockSpec((1, D), lambda j: (0, 0)),
            pl.BlockSpec((1, 91, D), wb_map),
            pl.BlockSpec((1, 3, D), ws_map),
            pl.BlockSpec((1, 91, 1), wb_map),
            pl.BlockSpec((1, 1, D), wb_map),
            pl.BlockSpec((1, 3, 1), ws_map),
            pl.BlockSpec((1, 1, D), ws_map),
        ],
        out_specs=pl.BlockSpec((1, D), lambda j: (0, 0)),
        out_shape=jax.ShapeDtypeStruct((1, D), jnp.float32),
        scratch_shapes=[
            pltpu.VMEM((1, D), jnp.float32),
            pltpu.VMEM((1, D), jnp.float32),
            pltpu.VMEM((1, D), jnp.float32),
        ],
    )(sample, wbT, wsT, fbT, bb, fsT, bs)

    return out2d.reshape(-1)


# speedup vs baseline: 1.0488x; 1.0268x over previous
import jax
import jax.numpy as jnp
from jax.experimental import pallas as pl
from jax.experimental.pallas import tpu as pltpu

NGRAM = 3
C = 4
LEVELS = 256
D = 8192
T = 1024
TB = 256
NTB = T // TB

_BIG_USED = (0, 2, 3, 4)
_SMALL_USED = (0, 4, 5, 6, 3, 17, 11, 12, 15)


def _prebind_kernel(lw_ref, keys_ref, m_ref):
    lw = lw_ref[...]
    for c in range(C):
        m_ref[c * LEVELS:(c + 1) * LEVELS, :] = (
            lw * keys_ref[c][None, :]).astype(jnp.int8)


def _stageA_kernel(sig_ref, m_ref, out_ref, prev2_ref):
    i = pl.program_id(0)
    idx = jnp.clip(jnp.round(sig_ref[...] * (LEVELS - 1)).astype(jnp.int32),
                   0, LEVELS - 1)
    iota_l = jax.lax.broadcasted_iota(jnp.int32, (TB, LEVELS), 1)
    onehot = jnp.concatenate(
        [(idx[:, c][:, None] == iota_l) for c in range(C)],
        axis=1).astype(jnp.int8)
    acc = jax.lax.dot_general(onehot, m_ref[...],
                              (((1,), (0,)), ((), ())),
                              preferred_element_type=jnp.int32)
    acc = acc.astype(jnp.bfloat16)
    rows = jnp.concatenate([prev2_ref[...], acc], axis=0)
    a = rows[0:TB]
    b = rows[1:TB + 1]
    cc = rows[2:TB + 2]
    a2 = jnp.concatenate([a[:, -2:], a[:, :-2]], axis=1)
    b1 = jnp.concatenate([b[:, -1:], b[:, :-1]], axis=1)
    prod = (a2 * b1 * cc).astype(jnp.float32)
    nskip = jnp.where(i == 0, 2, 0)
    riota = jax.lax.broadcasted_iota(jnp.int32, (TB, D), 0)
    prod = jnp.where(riota >= nskip, prod, 0.0)
    part = jnp.sum(prod, axis=0, keepdims=True)

    @pl.when(i == 0)
    def _():
        out_ref[...] = jnp.zeros_like(out_ref)

    out_ref[...] += part
    prev2_ref[...] = acc[TB - 2:TB]


def _stageB_kernel(sample_ref, wb_ref, ws_ref, fb_ref, bb_ref, fs_ref, bs_ref,
                   out_ref, mprod_ref, sa_ref, sb_ref):
    j = pl.program_id(0)

    @pl.when(j == 0)
    def _():
        mprod_ref[...] = jnp.ones_like(mprod_ref)
        sa_ref[...] = jnp.zeros_like(sa_ref)
        sb_ref[...] = jnp.zeros_like(sb_ref)

    def hv(w, fcol, brow):
        wb = w.astype(jnp.float32)
        fc = fcol.astype(jnp.bfloat16).astype(jnp.float32)
        p = jnp.sum(wb * fc, axis=0)[None, :]
        return jnp.cos(p + brow) * jnp.sin(p)

    @pl.when(j < 4)
    def _():
        mprod_ref[...] *= hv(wb_ref[0], fb_ref[0], bb_ref[0])

    @pl.when(jnp.logical_and(j >= 4, j < 8))
    def _():
        sa_ref[...] += hv(ws_ref[0], fs_ref[0], bs_ref[0])

    @pl.when(jnp.logical_and(j >= 8, j < 12))
    def _():
        sb_ref[...] += hv(ws_ref[0], fs_ref[0], bs_ref[0])

    @pl.when(j == 12)
    def _():
        h21 = hv(ws_ref[0], fs_ref[0], bs_ref[0])
        mult = mprod_ref[...] * sa_ref[...] * sb_ref[...] * h21
        s = sample_ref[...] * mult
        out_ref[...] = jnp.where(s > 0, 1.0, -1.0).astype(jnp.float32)


def kernel(signals, feat, keys_hv, level_weight, W_big, b_big, W_small, b_small):
    lw = level_weight.astype(jnp.bfloat16)
    keys = keys_hv.astype(jnp.bfloat16)

    m_tab = pl.pallas_call(
        _prebind_kernel,
        in_specs=[
            pl.BlockSpec((LEVELS, D), lambda: (0, 0)),
            pl.BlockSpec((C, D), lambda: (0, 0)),
        ],
        out_specs=pl.BlockSpec((C * LEVELS, D), lambda: (0, 0)),
        out_shape=jax.ShapeDtypeStruct((C * LEVELS, D), jnp.int8),
    )(lw, keys)

    sample = pl.pallas_call(
        _stageA_kernel,
        grid=(NTB,),
        in_specs=[
            pl.BlockSpec((TB, C), lambda i: (i, 0)),
            pl.BlockSpec((C * LEVELS, D), lambda i: (0, 0)),
        ],
        out_specs=pl.BlockSpec((1, D), lambda i: (0, 0)),
        out_shape=jax.ShapeDtypeStruct((1, D), jnp.float32),
        scratch_shapes=[pltpu.VMEM((2, D), jnp.bfloat16)],
    )(signals, m_tab)

    bigsel = jnp.array(_BIG_USED)
    smallsel = jnp.array(_SMALL_USED)
    wbT = W_big[bigsel].astype(jnp.bfloat16).transpose(0, 2, 1)
    wsT = W_small[smallsel].astype(jnp.bfloat16).transpose(0, 2, 1)
    fbT = feat[:546].reshape(6, 1, 91)[bigsel].transpose(0, 2, 1)
    fsT = feat[546:600].reshape(18, 1, 3)[smallsel].transpose(0, 2, 1)
    bb = b_big[bigsel][:, None, :]
    bs = b_small[smallsel][:, None, :]

    def wb_map(j):
        return (jnp.minimum(j, 3), 0, 0)

    def ws_map(j):
        return (jnp.maximum(j - 4, 0), 0, 0)

    out2d = pl.pallas_call(
        _stageB_kernel,
        grid=(13,),
        in_specs=[
            pl.BlockSpec((1, D), lambda j: (0, 0)),
            pl.BlockSpec((1, 91, D), wb_map),
            pl.BlockSpec((1, 3, D), ws_map),
            pl.BlockSpec((1, 91, 1), wb_map),
            pl.BlockSpec((1, 1, D), wb_map),
            pl.BlockSpec((1, 3, 1), ws_map),
            pl.BlockSpec((1, 1, D), ws_map),
        ],
        out_specs=pl.BlockSpec((1, D), lambda j: (0, 0)),
        out_shape=jax.ShapeDtypeStruct((1, D), jnp.float32),
        scratch_shapes=[
            pltpu.VMEM((1, D), jnp.float32),
            pltpu.VMEM((1, D), jnp.float32),
            pltpu.VMEM((1, D), jnp.float32),
        ],
    )(sample, wbT, wsT, fbT, bb, fsT, bs)

    return out2d.reshape(-1)
